# baseline (speedup 1.0000x reference)
import jax
import jax.numpy as jnp
from jax.experimental import pallas as pl
from jax.experimental.pallas import tpu as pltpu

NUM_TOKENS = 16384
D_MODEL = 2048
NUM_EXPERTS = 16
TOP_K = 2
CHUNK = 512
RING = 4
NCHUNKS = NUM_TOKENS // CHUNK


def _body(x_hbm, wt_ref, b_ref, idx_ref, val_ref, bufs, sems):
    def mkdma(c, slot):
        return pltpu.make_async_copy(
            x_hbm.at[pl.ds(c * CHUNK, CHUNK), :], bufs.at[slot], sems.at[slot]
        )

    for c in range(RING):
        mkdma(c, c).start()

    def step(c, _):
        slot = jax.lax.rem(c, RING)
        mkdma(c, slot).wait()
        s = jnp.dot(bufs[slot], wt_ref[...], preferred_element_type=jnp.float32)
        nxt = c + RING

        @pl.when(nxt < NCHUNKS)
        def _():
            mkdma(nxt, slot).start()

        s = s + b_ref[...]
        lane = jax.lax.broadcasted_iota(jnp.int32, s.shape, 1)
        m1 = jnp.max(s, axis=1, keepdims=True)
        i1 = jnp.argmax(s, axis=1).astype(jnp.int32)
        s2 = jnp.where(lane == i1[:, None], -jnp.inf, s)
        m2 = jnp.max(s2, axis=1, keepdims=True)
        i2 = jnp.argmax(s2, axis=1).astype(jnp.int32)
        z = jnp.sum(jnp.exp(s - m1), axis=1, keepdims=True)
        v1 = 1.0 / z
        v2 = jnp.exp(m2 - m1) * v1
        row = pl.ds(c * CHUNK, CHUNK)
        idx_ref[row, :] = jnp.concatenate([i1[:, None], i2[:, None]], axis=1)
        val_ref[row, :] = jnp.concatenate([v1, v2], axis=1)
        return 0

    jax.lax.fori_loop(0, NCHUNKS, step, 0)


@jax.jit
def kernel(x, W, b):
    wt = W.T
    b2 = b.reshape(1, NUM_EXPERTS)
    idx, val = pl.pallas_call(
        _body,
        in_specs=[
            pl.BlockSpec(memory_space=pltpu.MemorySpace.HBM),
            pl.BlockSpec((D_MODEL, NUM_EXPERTS), lambda: (0, 0)),
            pl.BlockSpec((1, NUM_EXPERTS), lambda: (0, 0)),
        ],
        out_specs=[
            pl.BlockSpec((NUM_TOKENS, TOP_K), lambda: (0, 0)),
            pl.BlockSpec((NUM_TOKENS, TOP_K), lambda: (0, 0)),
        ],
        out_shape=[
            jax.ShapeDtypeStruct((NUM_TOKENS, TOP_K), jnp.int32),
            jax.ShapeDtypeStruct((NUM_TOKENS, TOP_K), jnp.float32),
        ],
        scratch_shapes=[
            pltpu.VMEM((RING, CHUNK, D_MODEL), jnp.float32),
            pltpu.SemaphoreType.DMA((RING,)),
        ],
    )(x, wt, b2)
    return (idx, val)
